# baseline (speedup 1.0000x reference)
import functools

import jax
import jax.numpy as jnp
from jax import lax
from jax.experimental import pallas as pl
from jax.experimental.pallas import tpu as pltpu
from jax.experimental.pallas import tpu_sc as plsc

N = 10000
E = 320000
NCORES = 2
NSUB = 16
NTILES = NCORES * NSUB
E_TILE = E // NTILES
K = 80
NCHUNK = E_TILE // K
ROWB = 80
BM = 2000


def _edge_body(h, cout, with_deg, *refs):
    hc = h * cout
    it = iter(refs)
    xw_hbm = next(it)
    src2_hbm = next(it)
    dst2_hbm = next(it)
    z2d = next(it)
    if h > 1:
        xu_hbm = next(it)
        cb_hbm = next(it)
    if with_deg:
        zN = next(it)
    agg_out = next(it)
    if with_deg:
        deg_out = next(it)
    src_tab = next(it)
    dst_tab = next(it)
    xw_rows = [next(it), next(it)]
    agg_sh = next(it)
    sems = [next(it), next(it)]
    ssems = [next(it), next(it)]
    if h > 1:
        xu_tab = next(it)
        msgs = [next(it), next(it)]
        c_tab = next(it)
    if with_deg:
        ones_b = next(it)
        deg_sh = next(it)

    cid = lax.axis_index("c")
    sid = lax.axis_index("s")
    w = cid * NSUB + sid
    rbase = sid * 640
    nch = jnp.where(sid == NSUB - 1, 5, 8)

    @pl.loop(0, nch)
    def _init(j):
        r0 = rbase + j * ROWB
        pltpu.sync_copy(z2d.at[pl.ds(r0, ROWB)], agg_sh.at[pl.ds(r0, ROWB)])
        if with_deg:
            pltpu.sync_copy(zN.at[pl.ds(r0, ROWB)], deg_sh.at[pl.ds(r0, ROWB)])

    pltpu.sync_copy(src2_hbm.at[pl.ds(w * NCHUNK, NCHUNK)], src_tab)
    pltpu.sync_copy(dst2_hbm.at[pl.ds(w * NCHUNK, NCHUNK)], dst_tab)
    if h > 1:
        pltpu.sync_copy(xu_hbm, xu_tab)
        pltpu.sync_copy(cb_hbm, c_tab)
    if with_deg:
        for j in range(K // 16):
            ones_b[pl.ds(j * 16, 16)] = jnp.full((16,), 1.0, jnp.float32)
    plsc.subcore_barrier()

    lane0 = lax.iota(jnp.int32, 16)

    def compute_msg(i, p):
        hp = h + 1
        iv = jnp.zeros((16,), jnp.int32) + i
        all_qs = []
        for g in range(K // 16):
            laneg = lane0 + g * 16
            s_idx = plsc.load_gather(src_tab, [iv, laneg])
            d_idx = plsc.load_gather(dst_tab, [iv, laneg])
            sf = s_idx * hp
            df = d_idx * hp
            ts = []
            for hh in range(h):
                xus = plsc.load_gather(xu_tab, [sf + hh])
                xud = plsc.load_gather(xu_tab, [df + hh])
                ts.append(xus - xud + c_tab[pl.ds(hh * 16, 16)])
            m = ts[0]
            for t in ts[1:]:
                m = jnp.maximum(m, t)
            es = [jnp.exp(t - m) for t in ts]
            ssum = es[0]
            for e in es[1:]:
                ssum = ssum + e
            r = 1.0 / ssum
            all_qs.append([e * r for e in es])
        for g in range(K // 16):
            qs = all_qs[g]
            for ee in range(16):
                e = g * 16 + ee
                bidx = jnp.zeros((16,), jnp.int32) + ee
                bq = [q[bidx] for q in qs]
                for cb in range(cout // 16):
                    acc = bq[0] * xw_rows[p][e, pl.ds(cb * 16, 16)]
                    for hh in range(1, h):
                        acc = acc + bq[hh] * xw_rows[p][
                            e, pl.ds(hh * cout + cb * 16, 16)]
                    msgs[p][e, pl.ds(cb * 16, 16)] = acc

    def scatter_src(p):
        return msgs[p] if h > 1 else xw_rows[p]

    def scatter_start(i, p):
        pltpu.async_copy(scatter_src(p), agg_sh.at[dst_tab.at[i]], ssems[p],
                         add=True)
        if with_deg:
            pltpu.async_copy(ones_b, deg_sh.at[dst_tab.at[i]], ssems[p],
                             add=True)

    def scatter_drain(i, p):
        pltpu.make_async_copy(scatter_src(p), agg_sh.at[dst_tab.at[i]],
                              ssems[p]).wait()
        if with_deg:
            pltpu.make_async_copy(ones_b, deg_sh.at[dst_tab.at[i]],
                                  ssems[p]).wait()

    pltpu.async_copy(xw_hbm.at[src_tab.at[0]], xw_rows[0], sems[0])

    @pl.loop(0, (NCHUNK - 1) // 2)
    def _pair(j):
        for p in range(2):
            i = 2 * j + p
            pltpu.make_async_copy(xw_hbm.at[src_tab.at[i]],
                                  xw_rows[p], sems[p]).wait()
            if h > 1:
                pltpu.async_copy(xw_hbm.at[src_tab.at[i + 1]],
                                 xw_rows[1 - p], sems[1 - p])
                if p == 0:
                    @pl.when(j >= 1)
                    def _(
                    ):
                        scatter_drain(i - 2, p)
                else:
                    @pl.when(j >= 1)
                    def _(
                    ):
                        scatter_drain(i - 2, p)
                compute_msg(i, p)
            else:
                if p == 0:
                    @pl.when(j >= 1)
                    def _(
                    ):
                        scatter_drain(i - 1, 1 - p)
                else:
                    scatter_drain(i - 1, 1 - p)
                pltpu.async_copy(xw_hbm.at[src_tab.at[i + 1]],
                                 xw_rows[1 - p], sems[1 - p])
            scatter_start(i, p)

    i_last = NCHUNK - 1
    pltpu.make_async_copy(xw_hbm.at[src_tab.at[i_last]],
                          xw_rows[0], sems[0]).wait()
    if h > 1:
        scatter_drain(i_last - 2, 0)
        compute_msg(i_last, 0)
    else:
        scatter_drain(i_last - 1, 1)
    scatter_start(i_last, 0)
    scatter_drain(i_last, 0)
    if h > 1:
        scatter_drain(i_last - 1, 1)

    plsc.subcore_barrier()

    @pl.loop(0, nch)
    def _writeback(j):
        r0 = rbase + j * ROWB
        pltpu.sync_copy(agg_sh.at[pl.ds(r0, ROWB)],
                        agg_out.at[cid, pl.ds(r0, ROWB)])
        if with_deg:
            pltpu.sync_copy(deg_sh.at[pl.ds(r0, ROWB)],
                            deg_out.at[pl.ds(cid * N + r0, ROWB)])


def _edge_stage(xw, src, dst, c, h, cout, with_deg):
    hc = h * cout
    out_type = [jax.ShapeDtypeStruct((NCORES, N, cout), jnp.float32)]
    if with_deg:
        out_type.append(jax.ShapeDtypeStruct((NCORES * N,), jnp.float32))
    scratch = [
        pltpu.VMEM((NCHUNK, K), jnp.int32),
        pltpu.VMEM((NCHUNK, K), jnp.int32),
        pltpu.VMEM((K, hc), jnp.float32),
        pltpu.VMEM((K, hc), jnp.float32),
        pltpu.VMEM_SHARED((N, cout), jnp.float32),
        pltpu.SemaphoreType.DMA,
        pltpu.SemaphoreType.DMA,
        pltpu.SemaphoreType.DMA,
        pltpu.SemaphoreType.DMA,
    ]
    if h > 1:
        scratch += [
            pltpu.VMEM((N * (h + 1),), jnp.float32),
            pltpu.VMEM((K, cout), jnp.float32),
            pltpu.VMEM((K, cout), jnp.float32),
            pltpu.VMEM((h * 16,), jnp.float32),
        ]
    if with_deg:
        scratch += [
            pltpu.VMEM((K,), jnp.float32),
            pltpu.VMEM_SHARED((N,), jnp.float32),
        ]
    mesh = plsc.VectorSubcoreMesh(core_axis_name="c", subcore_axis_name="s")
    body = functools.partial(_edge_body, h, cout, with_deg)
    kern = pl.kernel(body, out_type=out_type, mesh=mesh, scratch_types=scratch,
                     compiler_params=pltpu.CompilerParams(
                         needs_layout_passes=False,
                         use_tc_tiling_on_sc=False))
    inputs = [xw, src.reshape(E // K, K), dst.reshape(E // K, K),
              jnp.zeros((N, cout), jnp.float32)]
    if h > 1:
        inputs += [c[0].reshape(N * (h + 1)), c[1].reshape(h * 16)]
    if with_deg:
        inputs.append(jnp.zeros((N,), jnp.float32))
    return kern(*inputs)


def _mm_xw_xu(parts, W, u):
    cins = [p.shape[1] for p in parts]
    cin = sum(cins)
    hc = W.shape[1]
    h = u.shape[1]
    np_ = len(parts)

    def body(*refs):
        part_refs = refs[:np_]
        w_ref = refs[np_]
        u_ref = refs[np_ + 1]
        xw_ref = refs[np_ + 2]
        xu_ref = refs[np_ + 3]
        accw = jnp.zeros((BM, hc), jnp.float32)
        accu = jnp.zeros((BM, h), jnp.float32)
        o = 0
        for pr, c in zip(part_refs, cins):
            xb = pr[...]
            accw += jnp.dot(xb, w_ref[o:o + c, :],
                            preferred_element_type=jnp.float32)
            accu += jnp.dot(xb, u_ref[o:o + c, :],
                            preferred_element_type=jnp.float32)
            o += c
        xw_ref[...] = accw
        xu_ref[...] = accu

    in_specs = [pl.BlockSpec((BM, c), lambda i: (i, 0)) for c in cins]
    in_specs += [pl.BlockSpec((cin, hc), lambda i: (0, 0)),
                 pl.BlockSpec((cin, h), lambda i: (0, 0))]
    return pl.pallas_call(
        body,
        grid=(N // BM,),
        in_specs=in_specs,
        out_specs=[pl.BlockSpec((BM, hc), lambda i: (i, 0)),
                   pl.BlockSpec((BM, h), lambda i: (i, 0))],
        out_shape=[jax.ShapeDtypeStruct((N, hc), jnp.float32),
                   jax.ShapeDtypeStruct((N, h), jnp.float32)],
    )(*parts, W, u)


def _epi_mm(agg2, xw, qc, b, dd, first, h, cout, parts, Wn, un):
    real_parts = [q for q in parts if not isinstance(q, str)]
    np_ = len(real_parts)
    cins = [cout if isinstance(q, str) else q.shape[1] for q in parts]
    cin = sum(cins)
    hcn = Wn.shape[1]
    hn = un.shape[1] if un is not None else 0

    def body(*refs):
        it = iter(refs)
        agg2_ref = next(it)
        xw_ref = next(it)
        qc_ref = next(it)
        b_ref = next(it)
        dd_ref = next(it)
        part_refs = [next(it) for _ in range(np_)]
        wn_ref = next(it)
        un_ref = next(it) if un is not None else None
        x_ref = next(it)
        if first:
            dinv_ref = next(it)
        xwn_ref = next(it)
        xun_ref = next(it) if un is not None else None
        agg = agg2_ref[0] + agg2_ref[1]
        selfm = jnp.zeros((BM, cout), jnp.float32)
        for hh in range(h):
            selfm += qc_ref[0, hh] * xw_ref[:, hh * cout:(hh + 1) * cout]
        if first:
            deg = dd_ref[0] + dd_ref[1]
            dv = 1.0 / (deg + 1.0)
            dinv_ref[...] = dv
        else:
            dv = dd_ref[...]
        xb_new = jnp.maximum((agg + selfm) * dv + b_ref[...], 0.0)
        x_ref[...] = xb_new
        accw = jnp.zeros((BM, hcn), jnp.float32)
        accu = jnp.zeros((BM, hn), jnp.float32) if un is not None else None
        o = 0
        pi = 0
        for pspec, c in zip(parts, cins):
            if isinstance(pspec, str):
                xb = xb_new
            else:
                xb = part_refs[pi][...]
                pi += 1
            accw += jnp.dot(xb, wn_ref[o:o + c, :],
                            preferred_element_type=jnp.float32)
            if un is not None:
                accu += jnp.dot(xb, un_ref[o:o + c, :],
                                preferred_element_type=jnp.float32)
            o += c
        xwn_ref[...] = accw
        if un is not None:
            xun_ref[...] = accu

    in_specs = [
        pl.BlockSpec((NCORES, BM, cout), lambda i: (0, i, 0)),
        pl.BlockSpec((BM, h * cout), lambda i: (i, 0)),
        pl.BlockSpec((1, h), lambda i: (0, 0)),
        pl.BlockSpec((1, cout), lambda i: (0, 0)),
    ]
    if first:
        in_specs.append(pl.BlockSpec((NCORES, BM, 1), lambda i: (0, i, 0)))
    else:
        in_specs.append(pl.BlockSpec((BM, 1), lambda i: (i, 0)))
    in_specs += [pl.BlockSpec((BM, q.shape[1]), lambda i: (i, 0))
                 for q in real_parts]
    in_specs.append(pl.BlockSpec((cin, hcn), lambda i: (0, 0)))
    if un is not None:
        in_specs.append(pl.BlockSpec((cin, hn), lambda i: (0, 0)))
    out_specs = [pl.BlockSpec((BM, cout), lambda i: (i, 0))]
    out_shape = [jax.ShapeDtypeStruct((N, cout), jnp.float32)]
    if first:
        out_specs.append(pl.BlockSpec((BM, 1), lambda i: (i, 0)))
        out_shape.append(jax.ShapeDtypeStruct((N, 1), jnp.float32))
    out_specs.append(pl.BlockSpec((BM, hcn), lambda i: (i, 0)))
    out_shape.append(jax.ShapeDtypeStruct((N, hcn), jnp.float32))
    if un is not None:
        out_specs.append(pl.BlockSpec((BM, hn), lambda i: (i, 0)))
        out_shape.append(jax.ShapeDtypeStruct((N, hn), jnp.float32))
    args = [agg2, xw, qc, b, dd] + real_parts + [Wn]
    if un is not None:
        args.append(un)
    return pl.pallas_call(
        body,
        grid=(N // BM,),
        in_specs=in_specs,
        out_specs=out_specs,
        out_shape=out_shape,
    )(*args)


def _mlp(agg2, xw, qc, b, dinv, h, cout, parts, relu_mask,
         lw1, lb1, lw2, lb2, lw3, lb3, lw4, lb4):
    real_parts = [q for q in parts if not isinstance(q, str)]
    np_ = len(real_parts)
    cins = [cout if isinstance(q, str) else q.shape[1] for q in parts]

    def body(*refs):
        it = iter(refs)
        agg2_ref = next(it)
        xw_ref = next(it)
        qc_ref = next(it)
        b_ref = next(it)
        dinv_ref = next(it)
        part_refs = [next(it) for _ in range(np_)]
        lw1_r, lb1_r, lw2_r, lb2_r, lw3_r, lb3_r, lw4_r, lb4_r = \
            [next(it) for _ in range(8)]
        out_ref = next(it)
        agg = agg2_ref[0] + agg2_ref[1]
        selfm = jnp.zeros((BM, cout), jnp.float32)
        for hh in range(h):
            selfm += qc_ref[0, hh] * xw_ref[:, hh * cout:(hh + 1) * cout]
        xb_new = jnp.maximum((agg + selfm) * dinv_ref[...] + b_ref[...], 0.0)
        z = lb1_r[...] + jnp.zeros((BM, lw1.shape[1]), jnp.float32)
        o = 0
        pi = 0
        for pspec, c, rl in zip(parts, cins, relu_mask):
            if isinstance(pspec, str):
                xb = xb_new
            else:
                xb = part_refs[pi][...]
                pi += 1
                if rl:
                    xb = jnp.maximum(xb, 0.0)
            z += jnp.dot(xb, lw1_r[o:o + c, :],
                         preferred_element_type=jnp.float32)
            o += c
        z = jnp.maximum(z, 0.0)
        z = jnp.maximum(jnp.dot(z, lw2_r[...],
                                preferred_element_type=jnp.float32)
                        + lb2_r[...], 0.0)
        z = jnp.maximum(jnp.dot(z, lw3_r[...],
                                preferred_element_type=jnp.float32)
                        + lb3_r[...], 0.0)
        z = jnp.dot(z, lw4_r[...], preferred_element_type=jnp.float32) \
            + lb4_r[...]
        out_ref[...] = jax.nn.sigmoid(z)

    in_specs = [
        pl.BlockSpec((NCORES, BM, cout), lambda i: (0, i, 0)),
        pl.BlockSpec((BM, h * cout), lambda i: (i, 0)),
        pl.BlockSpec((1, h), lambda i: (0, 0)),
        pl.BlockSpec((1, cout), lambda i: (0, 0)),
        pl.BlockSpec((BM, 1), lambda i: (i, 0)),
    ]
    in_specs += [pl.BlockSpec((BM, q.shape[1]), lambda i: (i, 0))
                 for q in real_parts]
    for wgt in (lw1, lb1, lw2, lb2, lw3, lb3, lw4, lb4):
        in_specs.append(pl.BlockSpec(wgt.shape, lambda i: (0,) * wgt.ndim))
    return pl.pallas_call(
        body,
        grid=(N // BM,),
        in_specs=in_specs,
        out_specs=pl.BlockSpec((BM, 1), lambda i: (i, 0)),
        out_shape=jax.ShapeDtypeStruct((N, 1), jnp.float32),
    )(agg2, xw, qc, b, dinv, *real_parts,
      lw1, lb1, lw2, lb2, lw3, lb3, lw4, lb4)


def kernel(x, edge_index, W1, u1, c1, b1, W2, u2, c2, b2, W3, u3, c3, b3,
           W4, u4, c4, b4, W5, u5, c5, b5, W6, u6, c6, b6,
           lw1, lb1, lw2, lb2, lw3, lb3, lw4, lb4):
    src = edge_index[0]
    dst = edge_index[1]

    def padu(u):
        return jnp.concatenate([u, jnp.zeros((u.shape[0], 1), u.dtype)],
                               axis=1)

    cfgs = [(W1, u1, c1, b1, 16, 4), (W2, u2, c2, b2, 32, 4),
            (W3, u3, c3, b3, 32, 4), (W4, u4, c4, b4, 32, 1),
            (W5, u5, c5, b5, 32, 1), (W6, u6, c6, b6, 64, 1)]
    qcs = [jax.nn.softmax(c)[None, :] for (_, _, c, _, _, _) in cfgs]
    cbs = [jnp.broadcast_to(c[:, None], (hh, 16)).astype(jnp.float32)
           for (_, _, c, _, _, hh) in cfgs]

    xw, xu = _mm_xw_xu([x], W1, padu(u1))
    agg2, degf = _edge_stage(xw, src, dst, (xu, cbs[0]), 4, 16, True)
    x1, dinv, xw2, xu2 = _epi_mm(agg2, xw, qcs[0], b1[None, :],
                                 degf.reshape(NCORES, N, 1), True, 4, 16,
                                 ["new", x], W2, padu(u2))
    (agg2,) = _edge_stage(xw2, src, dst, (xu2, cbs[1]), 4, 32, False)
    x2, xw3, xu3 = _epi_mm(agg2, xw2, qcs[1], b2[None, :], dinv, False, 4, 32,
                           [x1, x, "new"], W3, padu(u3))
    (agg2,) = _edge_stage(xw3, src, dst, (xu3, cbs[2]), 4, 32, False)
    x3, xw4 = _epi_mm(agg2, xw3, qcs[2], b3[None, :], dinv, False, 4, 32,
                      [x1, x, x2, "new"], W4, None)
    (agg2,) = _edge_stage(xw4, src, dst, None, 1, 32, False)
    x4, xw5 = _epi_mm(agg2, xw4, qcs[3], b4[None, :], dinv, False, 1, 32,
                      [x1, x, x2, x3, "new"], W5, None)
    (agg2,) = _edge_stage(xw5, src, dst, None, 1, 32, False)
    x5, xw6 = _epi_mm(agg2, xw5, qcs[4], b5[None, :], dinv, False, 1, 32,
                      [x1, x, x2, x3, x4, "new"], W6, None)
    (agg2,) = _edge_stage(xw6, src, dst, None, 1, 64, False)
    parts = [x1, x, x2, x3, x4, x5, "new"]
    relu_mask = [False, True, False, False, False, False, False]
    return _mlp(agg2, xw6, qcs[5], b6[None, :], dinv, 1, 64, parts, relu_mask,
                lw1, lb1[None, :], lw2, lb2[None, :],
                lw3, lb3[None, :], lw4, lb4[None, :])


# speedup vs baseline: 1.0704x; 1.0704x over previous
import functools

import jax
import jax.numpy as jnp
from jax import lax
from jax.experimental import pallas as pl
from jax.experimental.pallas import tpu as pltpu
from jax.experimental.pallas import tpu_sc as plsc

N = 10000
E = 320000
NCORES = 2
NSUB = 16
NTILES = NCORES * NSUB
E_TILE = E // NTILES
K = 80
NCHUNK = E_TILE // K
ROWB = 80
BM = 2000


def _edge_body(h, cout, with_deg, ke, *refs):
    hc = h * cout
    nchunk = E_TILE // ke
    it = iter(refs)
    xw_hbm = next(it)
    src2_hbm = next(it)
    dst2_hbm = next(it)
    if h > 1:
        xu_hbm = next(it)
        cb_hbm = next(it)
    agg_out = next(it)
    if with_deg:
        deg_out = next(it)
    src_tab = next(it)
    dst_tab = next(it)
    xw_rows = [next(it), next(it)]
    agg_sh = next(it)
    zbuf = next(it)
    sems = [next(it), next(it)]
    ssems = [next(it), next(it)]
    if h > 1:
        xu_tab = next(it)
        msgs = [next(it), next(it)]
        c_tab = next(it)
    if with_deg:
        ones_b = next(it)
        deg_sh = next(it)
        zdeg = next(it)

    cid = lax.axis_index("c")
    sid = lax.axis_index("s")
    w = cid * NSUB + sid
    rbase = sid * 640
    nch = jnp.where(sid == NSUB - 1, 5, 8)

    z16 = jnp.zeros((16,), jnp.float32)
    for r in range(ROWB):
        for cb in range(cout // 16):
            zbuf[r, pl.ds(cb * 16, 16)] = z16
    if with_deg:
        for jz in range(ROWB // 16):
            zdeg[pl.ds(jz * 16, 16)] = z16
        for jz in range(ke // 16):
            ones_b[pl.ds(jz * 16, 16)] = jnp.full((16,), 1.0, jnp.float32)

    @pl.loop(0, nch)
    def _init(j):
        r0 = rbase + j * ROWB
        pltpu.sync_copy(zbuf, agg_sh.at[pl.ds(r0, ROWB)])
        if with_deg:
            pltpu.sync_copy(zdeg, deg_sh.at[pl.ds(r0, ROWB)])

    pltpu.sync_copy(src2_hbm.at[pl.ds(w * nchunk, nchunk)], src_tab)
    pltpu.sync_copy(dst2_hbm.at[pl.ds(w * nchunk, nchunk)], dst_tab)
    if h > 1:
        pltpu.sync_copy(xu_hbm, xu_tab)
        pltpu.sync_copy(cb_hbm, c_tab)
    plsc.subcore_barrier()

    lane0 = lax.iota(jnp.int32, 16)

    def compute_msg(i, p):
        hp = h + 1
        iv = jnp.zeros((16,), jnp.int32) + i
        all_qs = []
        for g in range(ke // 16):
            laneg = lane0 + g * 16
            s_idx = plsc.load_gather(src_tab, [iv, laneg])
            d_idx = plsc.load_gather(dst_tab, [iv, laneg])
            sf = s_idx * hp
            df = d_idx * hp
            ts = []
            for hh in range(h):
                xus = plsc.load_gather(xu_tab, [sf + hh])
                xud = plsc.load_gather(xu_tab, [df + hh])
                ts.append(xus - xud + c_tab[pl.ds(hh * 16, 16)])
            m = ts[0]
            for t in ts[1:]:
                m = jnp.maximum(m, t)
            es = [jnp.exp(t - m) for t in ts]
            ssum = es[0]
            for e in es[1:]:
                ssum = ssum + e
            r = 1.0 / ssum
            all_qs.append([e * r for e in es])
        for g in range(ke // 16):
            qs = all_qs[g]
            for ee in range(16):
                e = g * 16 + ee
                bidx = jnp.zeros((16,), jnp.int32) + ee
                bq = [q[bidx] for q in qs]
                for cb in range(cout // 16):
                    acc = bq[0] * xw_rows[p][e, pl.ds(cb * 16, 16)]
                    for hh in range(1, h):
                        acc = acc + bq[hh] * xw_rows[p][
                            e, pl.ds(hh * cout + cb * 16, 16)]
                    msgs[p][e, pl.ds(cb * 16, 16)] = acc

    def scatter_src(p):
        return msgs[p] if h > 1 else xw_rows[p]

    def scatter_start(i, p):
        pltpu.async_copy(scatter_src(p), agg_sh.at[dst_tab.at[i]], ssems[p],
                         add=True)
        if with_deg:
            pltpu.async_copy(ones_b, deg_sh.at[dst_tab.at[i]], ssems[p],
                             add=True)

    def scatter_drain(i, p):
        pltpu.make_async_copy(scatter_src(p), agg_sh.at[dst_tab.at[i]],
                              ssems[p]).wait()
        if with_deg:
            pltpu.make_async_copy(ones_b, deg_sh.at[dst_tab.at[i]],
                                  ssems[p]).wait()

    pltpu.async_copy(xw_hbm.at[src_tab.at[0]], xw_rows[0], sems[0])
    npairs = nchunk // 2 if nchunk % 2 == 0 else (nchunk - 1) // 2

    @pl.loop(0, npairs)
    def _pair(j):
        for p in range(2):
            i = 2 * j + p
            pltpu.make_async_copy(xw_hbm.at[src_tab.at[i]],
                                  xw_rows[p], sems[p]).wait()
            if h > 1:
                pltpu.async_copy(xw_hbm.at[src_tab.at[i + 1]],
                                 xw_rows[1 - p], sems[1 - p])

                @pl.when(j >= 1)
                def _drain():
                    scatter_drain(i - 2, p)
                compute_msg(i, p)
            else:
                if p == 0:
                    @pl.when(j >= 1)
                    def _drain():
                        scatter_drain(i - 1, 1 - p)
                else:
                    scatter_drain(i - 1, 1 - p)
                if nchunk % 2 == 0:
                    @pl.when(i + 1 < nchunk)
                    def _prefetch():
                        pltpu.async_copy(xw_hbm.at[src_tab.at[i + 1]],
                                         xw_rows[1 - p], sems[1 - p])
                else:
                    pltpu.async_copy(xw_hbm.at[src_tab.at[i + 1]],
                                     xw_rows[1 - p], sems[1 - p])
            scatter_start(i, p)

    if nchunk % 2 == 1:
        i_last = nchunk - 1
        pltpu.make_async_copy(xw_hbm.at[src_tab.at[i_last]],
                              xw_rows[0], sems[0]).wait()
        if h > 1:
            scatter_drain(i_last - 2, 0)
            compute_msg(i_last, 0)
        else:
            scatter_drain(i_last - 1, 1)
        scatter_start(i_last, 0)
        scatter_drain(i_last, 0)
        scatter_drain(i_last - 1, 1)
    else:
        scatter_drain(nchunk - 1, 1)

    plsc.subcore_barrier()

    @pl.loop(0, nch)
    def _writeback(j):
        r0 = rbase + j * ROWB
        pltpu.sync_copy(agg_sh.at[pl.ds(r0, ROWB)],
                        agg_out.at[cid, pl.ds(r0, ROWB)])
        if with_deg:
            pltpu.sync_copy(deg_sh.at[pl.ds(r0, ROWB)],
                            deg_out.at[pl.ds(cid * N + r0, ROWB)])


def _edge_stage(xw, src, dst, c, h, cout, with_deg):
    hc = h * cout
    ke = 80 if h > 1 else 100
    nchunk = E_TILE // ke
    out_type = [jax.ShapeDtypeStruct((NCORES, N, cout), jnp.float32)]
    if with_deg:
        out_type.append(jax.ShapeDtypeStruct((NCORES * N,), jnp.float32))
    scratch = [
        pltpu.VMEM((nchunk, ke), jnp.int32),
        pltpu.VMEM((nchunk, ke), jnp.int32),
        pltpu.VMEM((ke, hc), jnp.float32),
        pltpu.VMEM((ke, hc), jnp.float32),
        pltpu.VMEM_SHARED((N, cout), jnp.float32),
        pltpu.VMEM((ROWB, cout), jnp.float32),
        pltpu.SemaphoreType.DMA,
        pltpu.SemaphoreType.DMA,
        pltpu.SemaphoreType.DMA,
        pltpu.SemaphoreType.DMA,
    ]
    if h > 1:
        scratch += [
            pltpu.VMEM((N * (h + 1),), jnp.float32),
            pltpu.VMEM((ke, cout), jnp.float32),
            pltpu.VMEM((ke, cout), jnp.float32),
            pltpu.VMEM((h * 16,), jnp.float32),
        ]
    if with_deg:
        scratch += [
            pltpu.VMEM((ke,), jnp.float32),
            pltpu.VMEM_SHARED((N,), jnp.float32),
            pltpu.VMEM((ROWB,), jnp.float32),
        ]
    mesh = plsc.VectorSubcoreMesh(core_axis_name="c", subcore_axis_name="s")
    body = functools.partial(_edge_body, h, cout, with_deg, ke)
    kern = pl.kernel(body, out_type=out_type, mesh=mesh, scratch_types=scratch,
                     compiler_params=pltpu.CompilerParams(
                         needs_layout_passes=False,
                         use_tc_tiling_on_sc=False))
    inputs = [xw, src.reshape(E // ke, ke), dst.reshape(E // ke, ke)]
    if h > 1:
        inputs += [c[0].reshape(N * (h + 1)), c[1].reshape(h * 16)]
    return kern(*inputs)


def _mm_xw_xu(parts, W, u):
    cins = [p.shape[1] for p in parts]
    cin = sum(cins)
    hc = W.shape[1]
    h = u.shape[1]
    np_ = len(parts)

    def body(*refs):
        part_refs = refs[:np_]
        w_ref = refs[np_]
        u_ref = refs[np_ + 1]
        xw_ref = refs[np_ + 2]
        xu_ref = refs[np_ + 3]
        accw = jnp.zeros((BM, hc), jnp.float32)
        accu = jnp.zeros((BM, h), jnp.float32)
        o = 0
        for pr, c in zip(part_refs, cins):
            xb = pr[...]
            accw += jnp.dot(xb, w_ref[o:o + c, :],
                            preferred_element_type=jnp.float32)
            accu += jnp.dot(xb, u_ref[o:o + c, :],
                            preferred_element_type=jnp.float32)
            o += c
        xw_ref[...] = accw
        xu_ref[...] = accu

    in_specs = [pl.BlockSpec((BM, c), lambda i: (i, 0)) for c in cins]
    in_specs += [pl.BlockSpec((cin, hc), lambda i: (0, 0)),
                 pl.BlockSpec((cin, h), lambda i: (0, 0))]
    return pl.pallas_call(
        body,
        grid=(N // BM,),
        in_specs=in_specs,
        out_specs=[pl.BlockSpec((BM, hc), lambda i: (i, 0)),
                   pl.BlockSpec((BM, h), lambda i: (i, 0))],
        out_shape=[jax.ShapeDtypeStruct((N, hc), jnp.float32),
                   jax.ShapeDtypeStruct((N, h), jnp.float32)],
    )(*parts, W, u)


def _epi_mm(agg2, xw, qc, b, dd, first, h, cout, parts, Wn, un):
    real_parts = [q for q in parts if not isinstance(q, str)]
    np_ = len(real_parts)
    cins = [cout if isinstance(q, str) else q.shape[1] for q in parts]
    cin = sum(cins)
    hcn = Wn.shape[1]
    hn = un.shape[1] if un is not None else 0

    def body(*refs):
        it = iter(refs)
        agg2_ref = next(it)
        xw_ref = next(it)
        qc_ref = next(it)
        b_ref = next(it)
        dd_ref = next(it)
        part_refs = [next(it) for _ in range(np_)]
        wn_ref = next(it)
        un_ref = next(it) if un is not None else None
        x_ref = next(it)
        if first:
            dinv_ref = next(it)
        xwn_ref = next(it)
        xun_ref = next(it) if un is not None else None
        agg = agg2_ref[0] + agg2_ref[1]
        selfm = jnp.zeros((BM, cout), jnp.float32)
        for hh in range(h):
            selfm += qc_ref[0, hh] * xw_ref[:, hh * cout:(hh + 1) * cout]
        if first:
            deg = dd_ref[0] + dd_ref[1]
            dv = 1.0 / (deg + 1.0)
            dinv_ref[...] = dv
        else:
            dv = dd_ref[...]
        xb_new = jnp.maximum((agg + selfm) * dv + b_ref[...], 0.0)
        x_ref[...] = xb_new
        accw = jnp.zeros((BM, hcn), jnp.float32)
        accu = jnp.zeros((BM, hn), jnp.float32) if un is not None else None
        o = 0
        pi = 0
        for pspec, c in zip(parts, cins):
            if isinstance(pspec, str):
                xb = xb_new
            else:
                xb = part_refs[pi][...]
                pi += 1
            accw += jnp.dot(xb, wn_ref[o:o + c, :],
                            preferred_element_type=jnp.float32)
            if un is not None:
                accu += jnp.dot(xb, un_ref[o:o + c, :],
                                preferred_element_type=jnp.float32)
            o += c
        xwn_ref[...] = accw
        if un is not None:
            xun_ref[...] = accu

    in_specs = [
        pl.BlockSpec((NCORES, BM, cout), lambda i: (0, i, 0)),
        pl.BlockSpec((BM, h * cout), lambda i: (i, 0)),
        pl.BlockSpec((1, h), lambda i: (0, 0)),
        pl.BlockSpec((1, cout), lambda i: (0, 0)),
    ]
    if first:
        in_specs.append(pl.BlockSpec((NCORES, BM, 1), lambda i: (0, i, 0)))
    else:
        in_specs.append(pl.BlockSpec((BM, 1), lambda i: (i, 0)))
    in_specs += [pl.BlockSpec((BM, q.shape[1]), lambda i: (i, 0))
                 for q in real_parts]
    in_specs.append(pl.BlockSpec((cin, hcn), lambda i: (0, 0)))
    if un is not None:
        in_specs.append(pl.BlockSpec((cin, hn), lambda i: (0, 0)))
    out_specs = [pl.BlockSpec((BM, cout), lambda i: (i, 0))]
    out_shape = [jax.ShapeDtypeStruct((N, cout), jnp.float32)]
    if first:
        out_specs.append(pl.BlockSpec((BM, 1), lambda i: (i, 0)))
        out_shape.append(jax.ShapeDtypeStruct((N, 1), jnp.float32))
    out_specs.append(pl.BlockSpec((BM, hcn), lambda i: (i, 0)))
    out_shape.append(jax.ShapeDtypeStruct((N, hcn), jnp.float32))
    if un is not None:
        out_specs.append(pl.BlockSpec((BM, hn), lambda i: (i, 0)))
        out_shape.append(jax.ShapeDtypeStruct((N, hn), jnp.float32))
    args = [agg2, xw, qc, b, dd] + real_parts + [Wn]
    if un is not None:
        args.append(un)
    return pl.pallas_call(
        body,
        grid=(N // BM,),
        in_specs=in_specs,
        out_specs=out_specs,
        out_shape=out_shape,
    )(*args)


def _mlp(agg2, xw, qc, b, dinv, h, cout, parts, relu_mask,
         lw1, lb1, lw2, lb2, lw3, lb3, lw4, lb4):
    real_parts = [q for q in parts if not isinstance(q, str)]
    np_ = len(real_parts)
    cins = [cout if isinstance(q, str) else q.shape[1] for q in parts]

    def body(*refs):
        it = iter(refs)
        agg2_ref = next(it)
        xw_ref = next(it)
        qc_ref = next(it)
        b_ref = next(it)
        dinv_ref = next(it)
        part_refs = [next(it) for _ in range(np_)]
        lw1_r, lb1_r, lw2_r, lb2_r, lw3_r, lb3_r, lw4_r, lb4_r = \
            [next(it) for _ in range(8)]
        out_ref = next(it)
        agg = agg2_ref[0] + agg2_ref[1]
        selfm = jnp.zeros((BM, cout), jnp.float32)
        for hh in range(h):
            selfm += qc_ref[0, hh] * xw_ref[:, hh * cout:(hh + 1) * cout]
        xb_new = jnp.maximum((agg + selfm) * dinv_ref[...] + b_ref[...], 0.0)
        z = lb1_r[...] + jnp.zeros((BM, lw1.shape[1]), jnp.float32)
        o = 0
        pi = 0
        for pspec, c, rl in zip(parts, cins, relu_mask):
            if isinstance(pspec, str):
                xb = xb_new
            else:
                xb = part_refs[pi][...]
                pi += 1
                if rl:
                    xb = jnp.maximum(xb, 0.0)
            z += jnp.dot(xb, lw1_r[o:o + c, :],
                         preferred_element_type=jnp.float32)
            o += c
        z = jnp.maximum(z, 0.0)
        z = jnp.maximum(jnp.dot(z, lw2_r[...],
                                preferred_element_type=jnp.float32)
                        + lb2_r[...], 0.0)
        z = jnp.maximum(jnp.dot(z, lw3_r[...],
                                preferred_element_type=jnp.float32)
                        + lb3_r[...], 0.0)
        z = jnp.dot(z, lw4_r[...], preferred_element_type=jnp.float32) \
            + lb4_r[...]
        out_ref[...] = jax.nn.sigmoid(z)

    in_specs = [
        pl.BlockSpec((NCORES, BM, cout), lambda i: (0, i, 0)),
        pl.BlockSpec((BM, h * cout), lambda i: (i, 0)),
        pl.BlockSpec((1, h), lambda i: (0, 0)),
        pl.BlockSpec((1, cout), lambda i: (0, 0)),
        pl.BlockSpec((BM, 1), lambda i: (i, 0)),
    ]
    in_specs += [pl.BlockSpec((BM, q.shape[1]), lambda i: (i, 0))
                 for q in real_parts]
    for wgt in (lw1, lb1, lw2, lb2, lw3, lb3, lw4, lb4):
        in_specs.append(pl.BlockSpec(wgt.shape, lambda i: (0,) * wgt.ndim))
    return pl.pallas_call(
        body,
        grid=(N // BM,),
        in_specs=in_specs,
        out_specs=pl.BlockSpec((BM, 1), lambda i: (i, 0)),
        out_shape=jax.ShapeDtypeStruct((N, 1), jnp.float32),
    )(agg2, xw, qc, b, dinv, *real_parts,
      lw1, lb1, lw2, lb2, lw3, lb3, lw4, lb4)


def kernel(x, edge_index, W1, u1, c1, b1, W2, u2, c2, b2, W3, u3, c3, b3,
           W4, u4, c4, b4, W5, u5, c5, b5, W6, u6, c6, b6,
           lw1, lb1, lw2, lb2, lw3, lb3, lw4, lb4):
    src = edge_index[0]
    dst = edge_index[1]

    def padu(u):
        return jnp.concatenate([u, jnp.zeros((u.shape[0], 1), u.dtype)],
                               axis=1)

    cfgs = [(W1, u1, c1, b1, 16, 4), (W2, u2, c2, b2, 32, 4),
            (W3, u3, c3, b3, 32, 4), (W4, u4, c4, b4, 32, 1),
            (W5, u5, c5, b5, 32, 1), (W6, u6, c6, b6, 64, 1)]
    qcs = [jax.nn.softmax(c)[None, :] for (_, _, c, _, _, _) in cfgs]
    cbs = [jnp.broadcast_to(c[:, None], (hh, 16)).astype(jnp.float32)
           for (_, _, c, _, _, hh) in cfgs]

    xw, xu = _mm_xw_xu([x], W1, padu(u1))
    agg2, degf = _edge_stage(xw, src, dst, (xu, cbs[0]), 4, 16, True)
    x1, dinv, xw2, xu2 = _epi_mm(agg2, xw, qcs[0], b1[None, :],
                                 degf.reshape(NCORES, N, 1), True, 4, 16,
                                 ["new", x], W2, padu(u2))
    (agg2,) = _edge_stage(xw2, src, dst, (xu2, cbs[1]), 4, 32, False)
    x2, xw3, xu3 = _epi_mm(agg2, xw2, qcs[1], b2[None, :], dinv, False, 4, 32,
                           [x1, x, "new"], W3, padu(u3))
    (agg2,) = _edge_stage(xw3, src, dst, (xu3, cbs[2]), 4, 32, False)
    x3, xw4 = _epi_mm(agg2, xw3, qcs[2], b3[None, :], dinv, False, 4, 32,
                      [x1, x, x2, "new"], W4, None)
    (agg2,) = _edge_stage(xw4, src, dst, None, 1, 32, False)
    x4, xw5 = _epi_mm(agg2, xw4, qcs[3], b4[None, :], dinv, False, 1, 32,
                      [x1, x, x2, x3, "new"], W5, None)
    (agg2,) = _edge_stage(xw5, src, dst, None, 1, 32, False)
    x5, xw6 = _epi_mm(agg2, xw5, qcs[4], b5[None, :], dinv, False, 1, 32,
                      [x1, x, x2, x3, x4, "new"], W6, None)
    (agg2,) = _edge_stage(xw6, src, dst, None, 1, 64, False)
    parts = [x1, x, x2, x3, x4, x5, "new"]
    relu_mask = [False, True, False, False, False, False, False]
    return _mlp(agg2, xw6, qcs[5], b6[None, :], dinv, 1, 64, parts, relu_mask,
                lw1, lb1[None, :], lw2, lb2[None, :],
                lw3, lb3[None, :], lw4, lb4[None, :])
